# baseline (speedup 1.0000x reference)
import functools

import jax
import jax.numpy as jnp
from jax import lax
from jax.experimental import pallas as pl
from jax.experimental.pallas import tpu as pltpu
from jax.experimental.pallas import tpu_sc as plsc

B = 4096
CTX = 20
K = 20
D = 128
LANES = 16

NW = 32
PER_W = B // NW
G = 4
R = G * CTX
NCH = PER_W // G
DC = D // LANES
NGRP = (G * K) // LANES
H = R // 2
FLAT = PER_W * K
FPAD = FLAT + LANES
LN2 = 0.6931471805599453


def _xs(v, o):
    perm = lax.iota(jnp.int32, LANES) ^ o
    return jnp.take_along_axis(v, perm, axis=0)


def _butterfly16(vecs):
    iota = lax.iota(jnp.int32, LANES)
    o = 1
    while len(vecs) > 1:
        m = (iota & o) == 0
        vecs = [jnp.where(m, a + _xs(a, o), b + _xs(b, o))
                for a, b in zip(vecs[::2], vecs[1::2])]
        o *= 2
    return vecs[0]


def _compute_chunk(j, urows, vrows, srcacc, wbuf, ybuf, wlbuf):
    for e in range(G):
        base = e * CTX
        for dc in range(DC):
            off = dc * LANES
            acc = urows[base, pl.ds(off, LANES)]
            for c in range(1, CTX):
                acc = acc + urows[base + c, pl.ds(off, LANES)]
            srcacc[e, pl.ds(off, LANES)] = acc
    for g in range(NGRP):
        t0 = g * LANES
        partials = [jnp.zeros((LANES,), jnp.float32)] * LANES
        for dc in range(DC):
            off = dc * LANES
            ea = t0 // K
            eb = (t0 + LANES - 1) // K
            la = srcacc[ea, pl.ds(off, LANES)]
            lb = la if eb == ea else srcacc[eb, pl.ds(off, LANES)]
            for i in range(LANES):
                a = la if (t0 + i) // K == ea else lb
                partials[i] = partials[i] + a * vrows[t0 + i, pl.ds(off, LANES)]
        f = _butterfly16(partials)
        wv = wbuf[pl.ds(j * R + t0, LANES)]
        yv = ybuf[pl.ds(j * R + t0, LANES)]
        t = jnp.abs(f)
        t2 = t * t
        softplus = LN2 - 0.5 * t + 0.125 * t2 - (1.0 / 192.0) * t2 * t2
        bce = jnp.maximum(f, 0.0) - f * yv + softplus
        wlbuf[pl.ds(j * R + t0, LANES)] = wv * bce


def _sc_loss_kernel(src_idx, trg_idx, w_hbm, y_hbm, u_emb, v_emb, out,
                    idx_u, idx_v, u0, v0, u1, v1, srcacc,
                    wbuf, ybuf, wlbuf, accbuf,
                    su0, sv0, su1, sv1, tu0, tv0, tu1, tv1):
    wid = lax.axis_index("s") * 2 + lax.axis_index("c")

    pltpu.sync_copy(src_idx.at[wid], idx_u)
    pltpu.sync_copy(trg_idx.at[wid], idx_v)
    pltpu.sync_copy(w_hbm.at[wid], wbuf.at[pl.ds(0, FLAT)])
    pltpu.sync_copy(y_hbm.at[wid], ybuf.at[pl.ds(0, FLAT)])

    def issue(jc, ub, vb, su, sv, tu, tv):
        pltpu.async_copy(u_emb.at[idx_u.at[jc, pl.ds(0, H)]], ub.at[pl.ds(0, H)], su)
        pltpu.async_copy(v_emb.at[idx_v.at[jc, pl.ds(0, H)]], vb.at[pl.ds(0, H)], sv)
        pltpu.async_copy(u_emb.at[idx_u.at[jc, pl.ds(H, H)]], ub.at[pl.ds(H, H)], tu)
        pltpu.async_copy(v_emb.at[idx_v.at[jc, pl.ds(H, H)]], vb.at[pl.ds(H, H)], tv)

    def wait(ub, vb, su, sv, tu, tv):
        pltpu.make_async_copy(u_emb.at[pl.ds(0, H)], ub.at[pl.ds(0, H)], su).wait()
        pltpu.make_async_copy(v_emb.at[pl.ds(0, H)], vb.at[pl.ds(0, H)], sv).wait()
        pltpu.make_async_copy(u_emb.at[pl.ds(0, H)], ub.at[pl.ds(H, H)], tu).wait()
        pltpu.make_async_copy(v_emb.at[pl.ds(0, H)], vb.at[pl.ds(H, H)], tv).wait()

    issue(0, u0, v0, su0, sv0, tu0, tv0)

    def pair_body(i, carry):
        j0 = 2 * i
        issue(j0 + 1, u1, v1, su1, sv1, tu1, tv1)
        wait(u0, v0, su0, sv0, tu0, tv0)
        _compute_chunk(j0, u0, v0, srcacc, wbuf, ybuf, wlbuf)
        issue(jnp.minimum(j0 + 2, NCH - 1), u0, v0, su0, sv0, tu0, tv0)
        wait(u1, v1, su1, sv1, tu1, tv1)
        _compute_chunk(j0 + 1, u1, v1, srcacc, wbuf, ybuf, wlbuf)
        return carry

    lax.fori_loop(0, NCH // 2, pair_body, 0)
    wait(u0, v0, su0, sv0, tu0, tv0)

    iota = lax.iota(jnp.int32, LANES)
    smask = iota < (K - LANES)

    def batch_body(b8, acc):
        base = b8 * (LANES * K)

        def seg16(buf):
            main = [buf[pl.ds(base + K * l, LANES)] for l in range(LANES)]
            stragglers = [
                jnp.where(smask, buf[pl.ds(base + K * l + LANES, LANES)], 0.0)
                for l in range(LANES)
            ]
            return _butterfly16(main) + _butterfly16(stragglers)

        return acc + seg16(wlbuf) / seg16(wbuf)

    acc = lax.fori_loop(0, PER_W // LANES, batch_body,
                        jnp.zeros((LANES,), jnp.float32))
    accbuf[...] = acc
    pltpu.sync_copy(accbuf, out.at[wid])


def _sc_loss(src_words, trg_words, wmasks, labels, u_embeddings, v_embeddings):
    mesh = plsc.VectorSubcoreMesh(core_axis_name="c", subcore_axis_name="s")
    kern = functools.partial(
        pl.kernel,
        mesh=mesh,
        out_type=jax.ShapeDtypeStruct((NW, LANES), jnp.float32),
        scratch_types=[
            pltpu.VMEM((NCH, R), jnp.int32),
            pltpu.VMEM((NCH, R), jnp.int32),
            pltpu.VMEM((R, D), jnp.float32),
            pltpu.VMEM((R, D), jnp.float32),
            pltpu.VMEM((R, D), jnp.float32),
            pltpu.VMEM((R, D), jnp.float32),
            pltpu.VMEM((G, D), jnp.float32),
            pltpu.VMEM((FPAD,), jnp.float32),
            pltpu.VMEM((FPAD,), jnp.float32),
            pltpu.VMEM((FPAD,), jnp.float32),
            pltpu.VMEM((LANES,), jnp.float32),
            pltpu.SemaphoreType.DMA,
            pltpu.SemaphoreType.DMA,
            pltpu.SemaphoreType.DMA,
            pltpu.SemaphoreType.DMA,
            pltpu.SemaphoreType.DMA,
            pltpu.SemaphoreType.DMA,
            pltpu.SemaphoreType.DMA,
            pltpu.SemaphoreType.DMA,
        ],
    )(_sc_loss_kernel)
    src_idx = src_words.reshape(NW, NCH, R)
    trg_idx = trg_words.reshape(NW, NCH, R)
    w3 = wmasks.reshape(NW, FLAT)
    y3 = labels.reshape(NW, FLAT)
    return kern(src_idx, trg_idx, w3, y3, u_embeddings, v_embeddings)


def _mean_kernel(x_ref, out_ref):
    out_ref[0, 0] = jnp.sum(x_ref[...]) * (1.0 / B)


def kernel(src_words, trg_words, wmasks, labels, u_embeddings, v_embeddings):
    partial_sums = _sc_loss(src_words, trg_words, wmasks, labels,
                            u_embeddings, v_embeddings)
    loss = pl.pallas_call(
        _mean_kernel,
        out_shape=jax.ShapeDtypeStruct((1, 1), jnp.float32),
        out_specs=pl.BlockSpec(memory_space=pltpu.SMEM),
    )(partial_sums)
    return loss.reshape(())


# speedup vs baseline: 1.1772x; 1.1772x over previous
import functools

import jax
import jax.numpy as jnp
from jax import lax
from jax.experimental import pallas as pl
from jax.experimental.pallas import tpu as pltpu
from jax.experimental.pallas import tpu_sc as plsc

B = 4096
CTX = 20
K = 20
D = 128
LANES = 16

NW = 32
PER_W = B // NW
G = 4
R = G * CTX
NCH = PER_W // G
DC = D // LANES
NGRP = (G * K) // LANES


def _xs(v, o):
    perm = lax.iota(jnp.int32, LANES) ^ o
    return jnp.take_along_axis(v, perm, axis=0)


def _butterfly16(vecs):
    iota = lax.iota(jnp.int32, LANES)
    o = 1
    while len(vecs) > 1:
        m = (iota & o) == 0
        vecs = [jnp.where(m, a + _xs(a, o), b + _xs(b, o))
                for a, b in zip(vecs[::2], vecs[1::2])]
        o *= 2
    return vecs[0]


def _compute_chunk(j, urows, vrows, srcacc, pred):
    for e in range(G):
        base = e * CTX
        for dc in range(DC):
            off = dc * LANES
            acc = urows[base, pl.ds(off, LANES)]
            for c in range(1, CTX):
                acc = acc + urows[base + c, pl.ds(off, LANES)]
            srcacc[e, pl.ds(off, LANES)] = acc
    for g in range(NGRP):
        t0 = g * LANES
        partials = [jnp.zeros((LANES,), jnp.float32)] * LANES
        for dc in range(DC):
            off = dc * LANES
            ea = t0 // K
            eb = (t0 + LANES - 1) // K
            la = srcacc[ea, pl.ds(off, LANES)]
            lb = la if eb == ea else srcacc[eb, pl.ds(off, LANES)]
            for i in range(LANES):
                a = la if (t0 + i) // K == ea else lb
                partials[i] = partials[i] + a * vrows[t0 + i, pl.ds(off, LANES)]
        pred[j, pl.ds(t0, LANES)] = _butterfly16(partials)


H = R // 2


def _sc_pred_kernel(src_idx, trg_idx, u_emb, v_emb, out,
                    idx_u, idx_v, u0, v0, u1, v1, srcacc, pred,
                    su0, sv0, su1, sv1, tu0, tv0, tu1, tv1):
    wid = lax.axis_index("s") * 2 + lax.axis_index("c")

    pltpu.sync_copy(src_idx.at[wid], idx_u)
    pltpu.sync_copy(trg_idx.at[wid], idx_v)

    def issue(jc, ub, vb, su, sv, tu, tv):
        pltpu.async_copy(u_emb.at[idx_u.at[jc, pl.ds(0, H)]], ub.at[pl.ds(0, H)], su)
        pltpu.async_copy(v_emb.at[idx_v.at[jc, pl.ds(0, H)]], vb.at[pl.ds(0, H)], sv)
        pltpu.async_copy(u_emb.at[idx_u.at[jc, pl.ds(H, H)]], ub.at[pl.ds(H, H)], tu)
        pltpu.async_copy(v_emb.at[idx_v.at[jc, pl.ds(H, H)]], vb.at[pl.ds(H, H)], tv)

    def wait(ub, vb, su, sv, tu, tv):
        pltpu.make_async_copy(u_emb.at[pl.ds(0, H)], ub.at[pl.ds(0, H)], su).wait()
        pltpu.make_async_copy(v_emb.at[pl.ds(0, H)], vb.at[pl.ds(0, H)], sv).wait()
        pltpu.make_async_copy(u_emb.at[pl.ds(0, H)], ub.at[pl.ds(H, H)], tu).wait()
        pltpu.make_async_copy(v_emb.at[pl.ds(0, H)], vb.at[pl.ds(H, H)], tv).wait()

    issue(0, u0, v0, su0, sv0, tu0, tv0)

    def pair_body(i, carry):
        j0 = 2 * i
        issue(j0 + 1, u1, v1, su1, sv1, tu1, tv1)
        wait(u0, v0, su0, sv0, tu0, tv0)
        _compute_chunk(j0, u0, v0, srcacc, pred)
        issue(jnp.minimum(j0 + 2, NCH - 1), u0, v0, su0, sv0, tu0, tv0)
        wait(u1, v1, su1, sv1, tu1, tv1)
        _compute_chunk(j0 + 1, u1, v1, srcacc, pred)
        return carry

    lax.fori_loop(0, NCH // 2, pair_body, 0)
    wait(u0, v0, su0, sv0, tu0, tv0)
    pltpu.sync_copy(pred, out.at[wid])


def _sc_pred(src_words, trg_words, u_embeddings, v_embeddings):
    mesh = plsc.VectorSubcoreMesh(core_axis_name="c", subcore_axis_name="s")
    kern = functools.partial(
        pl.kernel,
        mesh=mesh,
        out_type=jax.ShapeDtypeStruct((NW, NCH, R), jnp.float32),
        scratch_types=[
            pltpu.VMEM((NCH, R), jnp.int32),
            pltpu.VMEM((NCH, R), jnp.int32),
            pltpu.VMEM((R, D), jnp.float32),
            pltpu.VMEM((R, D), jnp.float32),
            pltpu.VMEM((R, D), jnp.float32),
            pltpu.VMEM((R, D), jnp.float32),
            pltpu.VMEM((G, D), jnp.float32),
            pltpu.VMEM((NCH, R), jnp.float32),
            pltpu.SemaphoreType.DMA,
            pltpu.SemaphoreType.DMA,
            pltpu.SemaphoreType.DMA,
            pltpu.SemaphoreType.DMA,
            pltpu.SemaphoreType.DMA,
            pltpu.SemaphoreType.DMA,
            pltpu.SemaphoreType.DMA,
            pltpu.SemaphoreType.DMA,
        ],
    )(_sc_pred_kernel)
    src_idx = src_words.reshape(NW, NCH, R)
    trg_idx = trg_words.reshape(NW, NCH, R)
    return kern(src_idx, trg_idx, u_embeddings, v_embeddings)


def _loss_kernel(pred_ref, w_ref, y_ref, out_ref):
    p = pred_ref[...]
    w = w_ref[...]
    y = y_ref[...]
    bce = jnp.maximum(p, 0.0) - p * y + jnp.log1p(jnp.exp(-jnp.abs(p)))
    wl = w * bce
    num = jnp.sum(wl, axis=1)
    den = jnp.sum(w, axis=1)
    out_ref[0, 0] = jnp.mean(num / den)


def kernel(src_words, trg_words, wmasks, labels, u_embeddings, v_embeddings):
    pred = _sc_pred(src_words, trg_words, u_embeddings, v_embeddings)
    pred = pred.reshape(B, K)
    loss = pl.pallas_call(
        _loss_kernel,
        out_shape=jax.ShapeDtypeStruct((1, 1), jnp.float32),
        out_specs=pl.BlockSpec(memory_space=pltpu.SMEM),
    )(pred, wmasks, labels)
    return loss.reshape(())
